# baseline (speedup 1.0000x reference)
import functools

import jax
import jax.numpy as jnp
from jax import lax
from jax.experimental import pallas as pl
from jax.experimental.pallas import tpu as pltpu
from jax.experimental.pallas import tpu_sc as plsc

_N = 10000
_E = 320000
_H = 128
_R = 32
_NB = 8
_SUB = 16
_NL = 2

_NC, _NS = 1, 16
_NW = _NC * _NS
_C = 64
_NCHUNK = 40
_NSUP = 8
_SUPE = _C * _NCHUNK
_EPW = _SUPE * _NSUP
_EPAD = _NW * _EPW
_G = _H // 16
_NP = 10240
_RPT = _NP // _NS

_NBLK = 2000
_NT = _N // _NBLK


_RBLK = 8


def _xform_body(x_ref, w_ref, lw_ref, b_ref, t_ref, sl_ref):
    for rr in range(_RBLK):
        t_ref[rr] = jnp.dot(x_ref[...], w_ref[rr],
                            preferred_element_type=jnp.float32)

    @pl.when(pl.program_id(1) == 0)
    def _():
        sl_ref[...] = (
            jnp.dot(x_ref[...], lw_ref[...], preferred_element_type=jnp.float32)
            + b_ref[...]
        )


def _transform(x, wd, lw, b):
    return pl.pallas_call(
        _xform_body,
        grid=(_NT, _R // _RBLK),
        in_specs=[
            pl.BlockSpec((_NBLK, _H), lambda nt, r: (nt, 0)),
            pl.BlockSpec((_RBLK, _H, _H), lambda nt, r: (r, 0, 0)),
            pl.BlockSpec((_H, _H), lambda nt, r: (0, 0)),
            pl.BlockSpec((1, _H), lambda nt, r: (0, 0)),
        ],
        out_specs=[
            pl.BlockSpec((_RBLK, _NBLK, _H), lambda nt, r: (r, nt, 0)),
            pl.BlockSpec((_NBLK, _H), lambda nt, r: (nt, 0)),
        ],
        out_shape=[
            jax.ShapeDtypeStruct((_R, _N, _H), jnp.float32),
            jax.ShapeDtypeStruct((_N, _H), jnp.float32),
        ],
    )(x, wd, lw, b)


def _xform_fused_body(p_ref, slp_ref, w_ref, lw_ref, b_ref, t_ref, sl_ref,
                      xs_ref):
    @pl.when(pl.program_id(1) == 0)
    def _():
        acc = p_ref[0]
        for c in range(1, _NC):
            acc = acc + p_ref[c]
        xs_ref[...] = jnp.maximum(acc + slp_ref[...], 0.0)
        sl_ref[...] = (
            jnp.dot(xs_ref[...], lw_ref[...], preferred_element_type=jnp.float32)
            + b_ref[...]
        )

    for rr in range(_RBLK):
        t_ref[rr] = jnp.dot(xs_ref[...], w_ref[rr],
                            preferred_element_type=jnp.float32)


def _transform_fused(p, slp, wd, lw, b):
    return pl.pallas_call(
        _xform_fused_body,
        grid=(_NT, _R // _RBLK),
        in_specs=[
            pl.BlockSpec((_NC, _NBLK, _H), lambda nt, r: (0, nt, 0)),
            pl.BlockSpec((_NBLK, _H), lambda nt, r: (nt, 0)),
            pl.BlockSpec((_RBLK, _H, _H), lambda nt, r: (r, 0, 0)),
            pl.BlockSpec((_H, _H), lambda nt, r: (0, 0)),
            pl.BlockSpec((1, _H), lambda nt, r: (0, 0)),
        ],
        out_specs=[
            pl.BlockSpec((_RBLK, _NBLK, _H), lambda nt, r: (r, nt, 0)),
            pl.BlockSpec((_NBLK, _H), lambda nt, r: (nt, 0)),
        ],
        out_shape=[
            jax.ShapeDtypeStruct((_R, _N, _H), jnp.float32),
            jax.ShapeDtypeStruct((_N, _H), jnp.float32),
        ],
        scratch_shapes=[pltpu.VMEM((_NBLK, _H), jnp.float32)],
    )(p, slp, wd, lw, b)


def _combine_body(relu, p_ref, sl_ref, o_ref):
    v = p_ref[0]
    for c in range(1, _NC):
        v = v + p_ref[c]
    v = v + sl_ref[...]
    o_ref[...] = jnp.maximum(v, 0.0) if relu else v


def _combine(p, sl, relu):
    return pl.pallas_call(
        functools.partial(_combine_body, relu),
        grid=(_NT,),
        in_specs=[
            pl.BlockSpec((_NC, _NBLK, _H), lambda nt: (0, nt, 0)),
            pl.BlockSpec((_NBLK, _H), lambda nt: (nt, 0)),
        ],
        out_specs=pl.BlockSpec((_NBLK, _H), lambda nt: (nt, 0)),
        out_shape=jax.ShapeDtypeStruct((_N, _H), jnp.float32),
    )(p, sl)


def _bcast_lane(vec16, lane):
    return lax.gather(
        vec16, jnp.full((16, 1), lane, jnp.int32),
        lax.GatherDimensionNumbers(offset_dims=(), collapsed_slice_dims=(0,),
                                   start_index_map=(0,)),
        slice_sizes=(1,), mode=lax.GatherScatterMode.PROMISE_IN_BOUNDS)


def _edge_agg_body(t_hbm, ep_hbm, norm_hbm, out_hbm,
                   ep_v, norm_v, rows_v, acc, g0, g1, s0, s1):
    cid = lax.axis_index("c")
    sid = lax.axis_index("s")
    wid = sid * _NC + cid

    def _zrow(i, carry):
        for g in range(_G):
            rows_v[i, pl.ds(g * 16, 16)] = jnp.zeros((16,), jnp.float32)
        return carry

    lax.fori_loop(0, 2 * _C, _zrow, 0)
    row0 = sid * _RPT
    for k in range(_RPT // (2 * _C)):
        pltpu.sync_copy(rows_v, acc.at[pl.ds(row0 + k * 2 * _C, 2 * _C)])
    plsc.subcore_barrier()

    h0 = rows_v.at[pl.ds(0, _C)]
    h1 = rows_v.at[pl.ds(_C, _C)]

    def _scale(half_ref, c, carry_unused):
        def _sub(j, c2):
            jr = j * 16
            ng = norm_v[pl.ds(c * _C + jr, 16)]
            for lane in range(16):
                nv = _bcast_lane(ng, lane)
                for g in range(_G):
                    s = pl.ds(g * 16, 16)
                    half_ref[jr + lane, s] = half_ref[jr + lane, s] * nv
            return c2

        return lax.fori_loop(0, _C // 16, _sub, carry_unused)

    def _super(sup, carry_sup):
        pltpu.sync_copy(ep_hbm.at[wid * _NSUP + sup], ep_v)
        pltpu.sync_copy(
            norm_hbm.at[pl.ds((wid * _NSUP + sup) * _SUPE, _SUPE)], norm_v)
        pltpu.async_copy(t_hbm.at[ep_v.at[0, 0]], h0, g0)

        def _pair(k, carry):
            ca = 2 * k
            cb = 2 * k + 1
            pltpu.make_async_copy(t_hbm.at[ep_v.at[0, ca]], h0, g0).wait()
            _scale(h0, ca, 0)

            @pl.when(k > 0)
            def _():
                pltpu.make_async_copy(h1, acc.at[ep_v.at[1, ca - 1]], s1).wait()

            pltpu.async_copy(t_hbm.at[ep_v.at[0, cb]], h1, g1)
            pltpu.async_copy(h0, acc.at[ep_v.at[1, ca]], s0, add=True)
            pltpu.make_async_copy(t_hbm.at[ep_v.at[0, cb]], h1, g1).wait()
            _scale(h1, cb, 0)
            pltpu.make_async_copy(h0, acc.at[ep_v.at[1, ca]], s0).wait()

            @pl.when(k < _NCHUNK // 2 - 1)
            def _():
                pltpu.async_copy(t_hbm.at[ep_v.at[0, ca + 2]], h0, g0)

            pltpu.async_copy(h1, acc.at[ep_v.at[1, cb]], s1, add=True)
            return carry

        lax.fori_loop(0, _NCHUNK // 2, _pair, 0)
        pltpu.make_async_copy(
            h1, acc.at[ep_v.at[1, _NCHUNK - 1]], s1).wait()
        return carry_sup

    lax.fori_loop(0, _NSUP, _super, 0)

    plsc.subcore_barrier()
    pltpu.sync_copy(acc.at[pl.ds(row0, _RPT)],
                    out_hbm.at[cid, pl.ds(row0, _RPT)])


@functools.cache
def _edge_agg_kernel():
    mesh = plsc.VectorSubcoreMesh(core_axis_name="c", subcore_axis_name="s",
                                  num_cores=_NC, num_subcores=_NS)
    return pl.kernel(
        _edge_agg_body,
        out_type=jax.ShapeDtypeStruct((_NC, _NP, _H), jnp.float32),
        mesh=mesh,
        scratch_types=[
            pltpu.VMEM((2, _NCHUNK, _C), jnp.int32),
            pltpu.VMEM((_SUPE,), jnp.float32),
            pltpu.VMEM((2 * _C, _H), jnp.float32),
            pltpu.VMEM_SHARED((_NP, _H), jnp.float32),
            pltpu.SemaphoreType.DMA,
            pltpu.SemaphoreType.DMA,
            pltpu.SemaphoreType.DMA,
            pltpu.SemaphoreType.DMA,
        ],
    )


def kernel(h, edge_index, r, norm, embed, weights, loop_weights, biases):
    x = jnp.take(embed, h, axis=0)

    ii = jnp.arange(_NB)
    wd = jnp.zeros((_NL, _R, _NB, _SUB, _NB, _SUB), jnp.float32)
    wd = wd.at[:, :, ii, :, ii, :].set(jnp.transpose(weights, (2, 0, 1, 3, 4)))
    wd = wd.reshape(_NL, _R, _H, _H)

    pad = _EPAD - _E

    def _pad1(a, dtype):
        a = a.astype(dtype).reshape(-1)
        return jnp.concatenate([a, jnp.zeros((pad,), dtype)])

    gidx = _pad1(r.astype(jnp.int32) * _N + edge_index[0], jnp.int32)
    dst2 = _pad1(edge_index[1], jnp.int32)
    ep = jnp.stack([gidx.reshape(_NW * _NSUP, _NCHUNK, _C),
                    dst2.reshape(_NW * _NSUP, _NCHUNK, _C)], axis=1)
    norm2 = _pad1(norm, jnp.float32)

    t, sl = _transform(x, wd[0], loop_weights[0], biases[0].reshape(1, _H))
    p = _edge_agg_kernel()(t.reshape(_R * _N, _H), ep, norm2)
    for l in range(1, _NL):
        t, sl = _transform_fused(p, sl, wd[l], loop_weights[l],
                                 biases[l].reshape(1, _H))
        p = _edge_agg_kernel()(t.reshape(_R * _N, _H), ep, norm2)
    return _combine(p, sl, relu=False)


# speedup vs baseline: 1.3339x; 1.3339x over previous
import functools

import jax
import jax.numpy as jnp
from jax import lax
from jax.experimental import pallas as pl
from jax.experimental.pallas import tpu as pltpu
from jax.experimental.pallas import tpu_sc as plsc

_N = 10000
_E = 320000
_H = 128
_R = 32
_NB = 8
_SUB = 16
_NL = 2

_NC, _NS = 2, 16
_NW = _NC * _NS
_C = 64
_NCHUNK = 40
_SUPA = 3
_SUPB = 5
_SUPE = _C * _NCHUNK
_TSUP = _NS * (_SUPA + _SUPB)
_EPAD = _TSUP * _SUPE
_G = _H // 16
_NP = 10240
_RPT = _NP // _NS

_NBLK = 2000
_NT = _N // _NBLK


_RBLK = 8


def _xform_body(x_ref, w_ref, lw_ref, b_ref, t_ref, sl_ref):
    for rr in range(_RBLK):
        t_ref[rr] = jnp.dot(x_ref[...], w_ref[rr],
                            preferred_element_type=jnp.float32)

    @pl.when(pl.program_id(1) == 0)
    def _():
        sl_ref[...] = (
            jnp.dot(x_ref[...], lw_ref[...], preferred_element_type=jnp.float32)
            + b_ref[...]
        )


def _transform(x, wd, lw, b):
    return pl.pallas_call(
        _xform_body,
        grid=(_NT, _R // _RBLK),
        in_specs=[
            pl.BlockSpec((_NBLK, _H), lambda nt, r: (nt, 0)),
            pl.BlockSpec((_RBLK, _H, _H), lambda nt, r: (r, 0, 0)),
            pl.BlockSpec((_H, _H), lambda nt, r: (0, 0)),
            pl.BlockSpec((1, _H), lambda nt, r: (0, 0)),
        ],
        out_specs=[
            pl.BlockSpec((_RBLK, _NBLK, _H), lambda nt, r: (r, nt, 0)),
            pl.BlockSpec((_NBLK, _H), lambda nt, r: (nt, 0)),
        ],
        out_shape=[
            jax.ShapeDtypeStruct((_R, _N, _H), jnp.float32),
            jax.ShapeDtypeStruct((_N, _H), jnp.float32),
        ],
    )(x, wd, lw, b)


def _xform_fused_body(p_ref, slp_ref, w_ref, lw_ref, b_ref, t_ref, sl_ref,
                      xs_ref):
    @pl.when(pl.program_id(1) == 0)
    def _():
        acc = p_ref[0]
        for c in range(1, _NC):
            acc = acc + p_ref[c]
        xs_ref[...] = jnp.maximum(acc + slp_ref[...], 0.0)
        sl_ref[...] = (
            jnp.dot(xs_ref[...], lw_ref[...], preferred_element_type=jnp.float32)
            + b_ref[...]
        )

    for rr in range(_RBLK):
        t_ref[rr] = jnp.dot(xs_ref[...], w_ref[rr],
                            preferred_element_type=jnp.float32)


def _transform_fused(p, slp, wd, lw, b):
    return pl.pallas_call(
        _xform_fused_body,
        grid=(_NT, _R // _RBLK),
        in_specs=[
            pl.BlockSpec((_NC, _NBLK, _H), lambda nt, r: (0, nt, 0)),
            pl.BlockSpec((_NBLK, _H), lambda nt, r: (nt, 0)),
            pl.BlockSpec((_RBLK, _H, _H), lambda nt, r: (r, 0, 0)),
            pl.BlockSpec((_H, _H), lambda nt, r: (0, 0)),
            pl.BlockSpec((1, _H), lambda nt, r: (0, 0)),
        ],
        out_specs=[
            pl.BlockSpec((_RBLK, _NBLK, _H), lambda nt, r: (r, nt, 0)),
            pl.BlockSpec((_NBLK, _H), lambda nt, r: (nt, 0)),
        ],
        out_shape=[
            jax.ShapeDtypeStruct((_R, _N, _H), jnp.float32),
            jax.ShapeDtypeStruct((_N, _H), jnp.float32),
        ],
        scratch_shapes=[pltpu.VMEM((_NBLK, _H), jnp.float32)],
    )(p, slp, wd, lw, b)


def _combine_body(relu, p_ref, sl_ref, o_ref):
    v = p_ref[0]
    for c in range(1, _NC):
        v = v + p_ref[c]
    v = v + sl_ref[...]
    o_ref[...] = jnp.maximum(v, 0.0) if relu else v


def _combine(p, sl, relu):
    return pl.pallas_call(
        functools.partial(_combine_body, relu),
        grid=(_NT,),
        in_specs=[
            pl.BlockSpec((_NC, _NBLK, _H), lambda nt: (0, nt, 0)),
            pl.BlockSpec((_NBLK, _H), lambda nt: (nt, 0)),
        ],
        out_specs=pl.BlockSpec((_NBLK, _H), lambda nt: (nt, 0)),
        out_shape=jax.ShapeDtypeStruct((_N, _H), jnp.float32),
    )(p, sl)


def _bcast_lane(vec16, lane):
    return lax.gather(
        vec16, jnp.full((16, 1), lane, jnp.int32),
        lax.GatherDimensionNumbers(offset_dims=(), collapsed_slice_dims=(0,),
                                   start_index_map=(0,)),
        slice_sizes=(1,), mode=lax.GatherScatterMode.PROMISE_IN_BOUNDS)


def _edge_agg_body(t_hbm, ep_hbm, norm_hbm, out_hbm,
                   ep_v, norm_v, rows_v, acc, g0, g1, s0, s1):
    cid = lax.axis_index("c")
    sid = lax.axis_index("s")
    nsup = jnp.where(cid == 0, _SUPA, _SUPB)
    sup_base = cid * (_NS * _SUPA) + sid * nsup

    def _zrow(i, carry):
        for g in range(_G):
            rows_v[i, pl.ds(g * 16, 16)] = jnp.zeros((16,), jnp.float32)
        return carry

    lax.fori_loop(0, 2 * _C, _zrow, 0)
    row0 = sid * _RPT
    for k in range(_RPT // (2 * _C)):
        pltpu.sync_copy(rows_v, acc.at[pl.ds(row0 + k * 2 * _C, 2 * _C)])
    plsc.subcore_barrier()

    h0 = rows_v.at[pl.ds(0, _C)]
    h1 = rows_v.at[pl.ds(_C, _C)]

    def _scale(half_ref, c, carry_unused):
        def _sub(j, c2):
            jr = j * 16
            ng = norm_v[pl.ds(c * _C + jr, 16)]
            for lane in range(16):
                nv = _bcast_lane(ng, lane)
                for g in range(_G):
                    s = pl.ds(g * 16, 16)
                    half_ref[jr + lane, s] = half_ref[jr + lane, s] * nv
            return c2

        return lax.fori_loop(0, _C // 16, _sub, carry_unused)

    def _super(sup, carry_sup):
        pltpu.sync_copy(ep_hbm.at[sup_base + sup], ep_v)
        pltpu.sync_copy(
            norm_hbm.at[pl.ds((sup_base + sup) * _SUPE, _SUPE)], norm_v)
        pltpu.async_copy(t_hbm.at[ep_v.at[0, 0]], h0, g0)

        def _pair(k, carry):
            ca = 2 * k
            cb = 2 * k + 1
            pltpu.make_async_copy(t_hbm.at[ep_v.at[0, ca]], h0, g0).wait()
            _scale(h0, ca, 0)

            @pl.when(k > 0)
            def _():
                pltpu.make_async_copy(h1, acc.at[ep_v.at[1, ca - 1]], s1).wait()

            pltpu.async_copy(t_hbm.at[ep_v.at[0, cb]], h1, g1)
            pltpu.async_copy(h0, acc.at[ep_v.at[1, ca]], s0, add=True)
            pltpu.make_async_copy(t_hbm.at[ep_v.at[0, cb]], h1, g1).wait()
            _scale(h1, cb, 0)
            pltpu.make_async_copy(h0, acc.at[ep_v.at[1, ca]], s0).wait()

            @pl.when(k < _NCHUNK // 2 - 1)
            def _():
                pltpu.async_copy(t_hbm.at[ep_v.at[0, ca + 2]], h0, g0)

            pltpu.async_copy(h1, acc.at[ep_v.at[1, cb]], s1, add=True)
            return carry

        lax.fori_loop(0, _NCHUNK // 2, _pair, 0)
        pltpu.make_async_copy(
            h1, acc.at[ep_v.at[1, _NCHUNK - 1]], s1).wait()
        return carry_sup

    lax.fori_loop(0, nsup, _super, 0)

    plsc.subcore_barrier()
    pltpu.sync_copy(acc.at[pl.ds(row0, _RPT)],
                    out_hbm.at[cid, pl.ds(row0, _RPT)])


@functools.cache
def _edge_agg_kernel():
    mesh = plsc.VectorSubcoreMesh(core_axis_name="c", subcore_axis_name="s",
                                  num_cores=_NC, num_subcores=_NS)
    return pl.kernel(
        _edge_agg_body,
        out_type=jax.ShapeDtypeStruct((_NC, _NP, _H), jnp.float32),
        mesh=mesh,
        scratch_types=[
            pltpu.VMEM((2, _NCHUNK, _C), jnp.int32),
            pltpu.VMEM((_SUPE,), jnp.float32),
            pltpu.VMEM((2 * _C, _H), jnp.float32),
            pltpu.VMEM_SHARED((_NP, _H), jnp.float32),
            pltpu.SemaphoreType.DMA,
            pltpu.SemaphoreType.DMA,
            pltpu.SemaphoreType.DMA,
            pltpu.SemaphoreType.DMA,
        ],
    )


def kernel(h, edge_index, r, norm, embed, weights, loop_weights, biases):
    x = jnp.take(embed, h, axis=0)

    ii = jnp.arange(_NB)
    wd = jnp.zeros((_NL, _R, _NB, _SUB, _NB, _SUB), jnp.float32)
    wd = wd.at[:, :, ii, :, ii, :].set(jnp.transpose(weights, (2, 0, 1, 3, 4)))
    wd = wd.reshape(_NL, _R, _H, _H)

    pad = _EPAD - _E

    def _pad1(a, dtype):
        a = a.astype(dtype).reshape(-1)
        return jnp.concatenate([a, jnp.zeros((pad,), dtype)])

    gidx = _pad1(r.astype(jnp.int32) * _N + edge_index[0], jnp.int32)
    dst2 = _pad1(edge_index[1], jnp.int32)
    ep = jnp.stack([gidx.reshape(_TSUP, _NCHUNK, _C),
                    dst2.reshape(_TSUP, _NCHUNK, _C)], axis=1)
    norm2 = _pad1(norm, jnp.float32)

    t, sl = _transform(x, wd[0], loop_weights[0], biases[0].reshape(1, _H))
    p = _edge_agg_kernel()(t.reshape(_R * _N, _H), ep, norm2)
    for l in range(1, _NL):
        t, sl = _transform_fused(p, sl, wd[l], loop_weights[l],
                                 biases[l].reshape(1, _H))
        p = _edge_agg_kernel()(t.reshape(_R * _N, _H), ep, norm2)
    return _combine(p, sl, relu=False)


# speedup vs baseline: 1.5367x; 1.1520x over previous
import functools

import jax
import jax.numpy as jnp
from jax import lax
from jax.experimental import pallas as pl
from jax.experimental.pallas import tpu as pltpu
from jax.experimental.pallas import tpu_sc as plsc

_N = 10000
_E = 320000
_H = 128
_R = 32
_NB = 8
_SUB = 16
_NL = 2

_NC, _NS = 2, 16
_NW = _NC * _NS
_C = 64
_NCHUNK = 40
_SUPA = 5
_SUPB = 3
_SUPE = _C * _NCHUNK
_TSUP = _NS * (_SUPA + _SUPB)
_EPAD = _TSUP * _SUPE
_G = _H // 16
_NP = 10240
_RPT = _NP // _NS

_NBLK = 2000
_NT = _N // _NBLK


_RBLK = 8


def _xform_body(x_ref, w_ref, lw_ref, b_ref, t_ref, sl_ref):
    for rr in range(_RBLK):
        t_ref[rr] = jnp.dot(x_ref[...], w_ref[rr],
                            preferred_element_type=jnp.float32)

    @pl.when(pl.program_id(1) == 0)
    def _():
        sl_ref[...] = (
            jnp.dot(x_ref[...], lw_ref[...], preferred_element_type=jnp.float32)
            + b_ref[...]
        )


def _transform(x, wd, lw, b):
    return pl.pallas_call(
        _xform_body,
        grid=(_NT, _R // _RBLK),
        in_specs=[
            pl.BlockSpec((_NBLK, _H), lambda nt, r: (nt, 0)),
            pl.BlockSpec((_RBLK, _H, _H), lambda nt, r: (r, 0, 0)),
            pl.BlockSpec((_H, _H), lambda nt, r: (0, 0)),
            pl.BlockSpec((1, _H), lambda nt, r: (0, 0)),
        ],
        out_specs=[
            pl.BlockSpec((_RBLK, _NBLK, _H), lambda nt, r: (r, nt, 0)),
            pl.BlockSpec((_NBLK, _H), lambda nt, r: (nt, 0)),
        ],
        out_shape=[
            jax.ShapeDtypeStruct((_R, _N, _H), jnp.float32),
            jax.ShapeDtypeStruct((_N, _H), jnp.float32),
        ],
    )(x, wd, lw, b)


def _xform_fused_body(p_ref, slp_ref, w_ref, lw_ref, b_ref, t_ref, sl_ref,
                      xs_ref):
    @pl.when(pl.program_id(1) == 0)
    def _():
        acc = p_ref[0]
        for c in range(1, _NC):
            acc = acc + p_ref[c]
        xs_ref[...] = jnp.maximum(acc + slp_ref[...], 0.0)
        sl_ref[...] = (
            jnp.dot(xs_ref[...], lw_ref[...], preferred_element_type=jnp.float32)
            + b_ref[...]
        )

    for rr in range(_RBLK):
        t_ref[rr] = jnp.dot(xs_ref[...], w_ref[rr],
                            preferred_element_type=jnp.float32)


def _transform_fused(p, slp, wd, lw, b):
    return pl.pallas_call(
        _xform_fused_body,
        grid=(_NT, _R // _RBLK),
        in_specs=[
            pl.BlockSpec((_NC, _NBLK, _H), lambda nt, r: (0, nt, 0)),
            pl.BlockSpec((_NBLK, _H), lambda nt, r: (nt, 0)),
            pl.BlockSpec((_RBLK, _H, _H), lambda nt, r: (r, 0, 0)),
            pl.BlockSpec((_H, _H), lambda nt, r: (0, 0)),
            pl.BlockSpec((1, _H), lambda nt, r: (0, 0)),
        ],
        out_specs=[
            pl.BlockSpec((_RBLK, _NBLK, _H), lambda nt, r: (r, nt, 0)),
            pl.BlockSpec((_NBLK, _H), lambda nt, r: (nt, 0)),
        ],
        out_shape=[
            jax.ShapeDtypeStruct((_R, _N, _H), jnp.float32),
            jax.ShapeDtypeStruct((_N, _H), jnp.float32),
        ],
        scratch_shapes=[pltpu.VMEM((_NBLK, _H), jnp.float32)],
    )(p, slp, wd, lw, b)


def _combine_body(relu, p_ref, sl_ref, o_ref):
    v = p_ref[0]
    for c in range(1, _NC):
        v = v + p_ref[c]
    v = v + sl_ref[...]
    o_ref[...] = jnp.maximum(v, 0.0) if relu else v


def _combine(p, sl, relu):
    return pl.pallas_call(
        functools.partial(_combine_body, relu),
        grid=(_NT,),
        in_specs=[
            pl.BlockSpec((_NC, _NBLK, _H), lambda nt: (0, nt, 0)),
            pl.BlockSpec((_NBLK, _H), lambda nt: (nt, 0)),
        ],
        out_specs=pl.BlockSpec((_NBLK, _H), lambda nt: (nt, 0)),
        out_shape=jax.ShapeDtypeStruct((_N, _H), jnp.float32),
    )(p, sl)


def _bcast_lane(vec16, lane):
    return lax.gather(
        vec16, jnp.full((16, 1), lane, jnp.int32),
        lax.GatherDimensionNumbers(offset_dims=(), collapsed_slice_dims=(0,),
                                   start_index_map=(0,)),
        slice_sizes=(1,), mode=lax.GatherScatterMode.PROMISE_IN_BOUNDS)


def _edge_agg_body(t_hbm, ep_hbm, norm_hbm, out_hbm,
                   ep_v, norm_v, rows_v, acc, g0, g1, s0, s1):
    cid = lax.axis_index("c")
    sid = lax.axis_index("s")
    nsup = jnp.where(cid == 0, _SUPA, _SUPB)
    sup_base = cid * (_NS * _SUPA) + sid * nsup

    def _zrow(i, carry):
        for g in range(_G):
            rows_v[i, pl.ds(g * 16, 16)] = jnp.zeros((16,), jnp.float32)
        return carry

    lax.fori_loop(0, 2 * _C, _zrow, 0)
    row0 = sid * _RPT
    for k in range(_RPT // (2 * _C)):
        pltpu.sync_copy(rows_v, acc.at[pl.ds(row0 + k * 2 * _C, 2 * _C)])
    plsc.subcore_barrier()

    h0 = rows_v.at[pl.ds(0, _C)]
    h1 = rows_v.at[pl.ds(_C, _C)]

    def _scale(half_ref, c, carry_unused):
        def _sub(j, c2):
            jr = j * 16
            ng = norm_v[pl.ds(c * _C + jr, 16)]
            for lane in range(16):
                nv = _bcast_lane(ng, lane)
                for g in range(_G):
                    s = pl.ds(g * 16, 16)
                    half_ref[jr + lane, s] = half_ref[jr + lane, s] * nv
            return c2

        return lax.fori_loop(0, _C // 16, _sub, carry_unused)

    def _super(sup, carry_sup):
        pltpu.sync_copy(ep_hbm.at[sup_base + sup], ep_v)
        pltpu.sync_copy(
            norm_hbm.at[pl.ds((sup_base + sup) * _SUPE, _SUPE)], norm_v)
        pltpu.async_copy(t_hbm.at[ep_v.at[0, 0]], h0, g0)

        def _pair(k, carry):
            ca = 2 * k
            cb = 2 * k + 1
            pltpu.make_async_copy(t_hbm.at[ep_v.at[0, ca]], h0, g0).wait()
            _scale(h0, ca, 0)

            @pl.when(k > 0)
            def _():
                pltpu.make_async_copy(h1, acc.at[ep_v.at[1, ca - 1]], s1).wait()

            pltpu.async_copy(t_hbm.at[ep_v.at[0, cb]], h1, g1)
            pltpu.async_copy(h0, acc.at[ep_v.at[1, ca]], s0, add=True)
            pltpu.make_async_copy(t_hbm.at[ep_v.at[0, cb]], h1, g1).wait()
            _scale(h1, cb, 0)
            pltpu.make_async_copy(h0, acc.at[ep_v.at[1, ca]], s0).wait()

            @pl.when(k < _NCHUNK // 2 - 1)
            def _():
                pltpu.async_copy(t_hbm.at[ep_v.at[0, ca + 2]], h0, g0)

            pltpu.async_copy(h1, acc.at[ep_v.at[1, cb]], s1, add=True)
            return carry

        lax.fori_loop(0, _NCHUNK // 2, _pair, 0)
        pltpu.make_async_copy(
            h1, acc.at[ep_v.at[1, _NCHUNK - 1]], s1).wait()
        return carry_sup

    lax.fori_loop(0, nsup, _super, 0)

    plsc.subcore_barrier()
    pltpu.sync_copy(acc.at[pl.ds(row0, _RPT)],
                    out_hbm.at[cid, pl.ds(row0, _RPT)])


@functools.cache
def _edge_agg_kernel():
    mesh = plsc.VectorSubcoreMesh(core_axis_name="c", subcore_axis_name="s",
                                  num_cores=_NC, num_subcores=_NS)
    return pl.kernel(
        _edge_agg_body,
        out_type=jax.ShapeDtypeStruct((_NC, _NP, _H), jnp.float32),
        mesh=mesh,
        scratch_types=[
            pltpu.VMEM((2, _NCHUNK, _C), jnp.int32),
            pltpu.VMEM((_SUPE,), jnp.float32),
            pltpu.VMEM((2 * _C, _H), jnp.float32),
            pltpu.VMEM_SHARED((_NP, _H), jnp.float32),
            pltpu.SemaphoreType.DMA,
            pltpu.SemaphoreType.DMA,
            pltpu.SemaphoreType.DMA,
            pltpu.SemaphoreType.DMA,
        ],
    )


def kernel(h, edge_index, r, norm, embed, weights, loop_weights, biases):
    x = jnp.take(embed, h, axis=0)

    ii = jnp.arange(_NB)
    wd = jnp.zeros((_NL, _R, _NB, _SUB, _NB, _SUB), jnp.float32)
    wd = wd.at[:, :, ii, :, ii, :].set(jnp.transpose(weights, (2, 0, 1, 3, 4)))
    wd = wd.reshape(_NL, _R, _H, _H)

    pad = _EPAD - _E

    def _pad1(a, dtype):
        a = a.astype(dtype).reshape(-1)
        return jnp.concatenate([a, jnp.zeros((pad,), dtype)])

    gidx = _pad1(r.astype(jnp.int32) * _N + edge_index[0], jnp.int32)
    dst2 = _pad1(edge_index[1], jnp.int32)
    ep = jnp.stack([gidx.reshape(_TSUP, _NCHUNK, _C),
                    dst2.reshape(_TSUP, _NCHUNK, _C)], axis=1)
    norm2 = _pad1(norm, jnp.float32)

    t, sl = _transform(x, wd[0], loop_weights[0], biases[0].reshape(1, _H))
    p = _edge_agg_kernel()(t.reshape(_R * _N, _H), ep, norm2)
    for l in range(1, _NL):
        t, sl = _transform_fused(p, sl, wd[l], loop_weights[l],
                                 biases[l].reshape(1, _H))
        p = _edge_agg_kernel()(t.reshape(_R * _N, _H), ep, norm2)
    return _combine(p, sl, relu=False)


# speedup vs baseline: 1.6971x; 1.1044x over previous
import functools

import jax
import jax.numpy as jnp
from jax import lax
from jax.experimental import pallas as pl
from jax.experimental.pallas import tpu as pltpu
from jax.experimental.pallas import tpu_sc as plsc

_N = 10000
_E = 320000
_H = 128
_R = 32
_NB = 8
_SUB = 16
_NL = 2

_NC, _NS = 2, 16
_NW = _NC * _NS
_C = 64
_NCHUNK = 40
_SUPA = 6
_SUPB = 2
_SUPE = _C * _NCHUNK
_TSUP = _NS * (_SUPA + _SUPB)
_EPAD = _TSUP * _SUPE
_G = _H // 16
_NP = 10240
_RPT = _NP // _NS

_NBLK = 2000
_NT = _N // _NBLK


_RBLK = 8


def _xform_body(x_ref, w_ref, lw_ref, b_ref, t_ref, sl_ref):
    for rr in range(_RBLK):
        t_ref[rr] = jnp.dot(x_ref[...], w_ref[rr],
                            preferred_element_type=jnp.float32)

    @pl.when(pl.program_id(1) == 0)
    def _():
        sl_ref[...] = (
            jnp.dot(x_ref[...], lw_ref[...], preferred_element_type=jnp.float32)
            + b_ref[...]
        )


def _transform(x, wd, lw, b):
    return pl.pallas_call(
        _xform_body,
        grid=(_NT, _R // _RBLK),
        in_specs=[
            pl.BlockSpec((_NBLK, _H), lambda nt, r: (nt, 0)),
            pl.BlockSpec((_RBLK, _H, _H), lambda nt, r: (r, 0, 0)),
            pl.BlockSpec((_H, _H), lambda nt, r: (0, 0)),
            pl.BlockSpec((1, _H), lambda nt, r: (0, 0)),
        ],
        out_specs=[
            pl.BlockSpec((_RBLK, _NBLK, _H), lambda nt, r: (r, nt, 0)),
            pl.BlockSpec((_NBLK, _H), lambda nt, r: (nt, 0)),
        ],
        out_shape=[
            jax.ShapeDtypeStruct((_R, _N, _H), jnp.float32),
            jax.ShapeDtypeStruct((_N, _H), jnp.float32),
        ],
    )(x, wd, lw, b)


def _xform_fused_body(p_ref, slp_ref, w_ref, lw_ref, b_ref, t_ref, sl_ref,
                      xs_ref):
    @pl.when(pl.program_id(1) == 0)
    def _():
        acc = p_ref[0]
        for c in range(1, _NC):
            acc = acc + p_ref[c]
        xs_ref[...] = jnp.maximum(acc + slp_ref[...], 0.0)
        sl_ref[...] = (
            jnp.dot(xs_ref[...], lw_ref[...], preferred_element_type=jnp.float32)
            + b_ref[...]
        )

    for rr in range(_RBLK):
        t_ref[rr] = jnp.dot(xs_ref[...], w_ref[rr],
                            preferred_element_type=jnp.float32)


def _transform_fused(p, slp, wd, lw, b):
    return pl.pallas_call(
        _xform_fused_body,
        grid=(_NT, _R // _RBLK),
        in_specs=[
            pl.BlockSpec((_NC, _NBLK, _H), lambda nt, r: (0, nt, 0)),
            pl.BlockSpec((_NBLK, _H), lambda nt, r: (nt, 0)),
            pl.BlockSpec((_RBLK, _H, _H), lambda nt, r: (r, 0, 0)),
            pl.BlockSpec((_H, _H), lambda nt, r: (0, 0)),
            pl.BlockSpec((1, _H), lambda nt, r: (0, 0)),
        ],
        out_specs=[
            pl.BlockSpec((_RBLK, _NBLK, _H), lambda nt, r: (r, nt, 0)),
            pl.BlockSpec((_NBLK, _H), lambda nt, r: (nt, 0)),
        ],
        out_shape=[
            jax.ShapeDtypeStruct((_R, _N, _H), jnp.float32),
            jax.ShapeDtypeStruct((_N, _H), jnp.float32),
        ],
        scratch_shapes=[pltpu.VMEM((_NBLK, _H), jnp.float32)],
    )(p, slp, wd, lw, b)


def _combine_body(relu, p_ref, sl_ref, o_ref):
    v = p_ref[0]
    for c in range(1, _NC):
        v = v + p_ref[c]
    v = v + sl_ref[...]
    o_ref[...] = jnp.maximum(v, 0.0) if relu else v


def _combine(p, sl, relu):
    return pl.pallas_call(
        functools.partial(_combine_body, relu),
        grid=(_NT,),
        in_specs=[
            pl.BlockSpec((_NC, _NBLK, _H), lambda nt: (0, nt, 0)),
            pl.BlockSpec((_NBLK, _H), lambda nt: (nt, 0)),
        ],
        out_specs=pl.BlockSpec((_NBLK, _H), lambda nt: (nt, 0)),
        out_shape=jax.ShapeDtypeStruct((_N, _H), jnp.float32),
    )(p, sl)


def _bcast_lane(vec16, lane):
    return lax.gather(
        vec16, jnp.full((16, 1), lane, jnp.int32),
        lax.GatherDimensionNumbers(offset_dims=(), collapsed_slice_dims=(0,),
                                   start_index_map=(0,)),
        slice_sizes=(1,), mode=lax.GatherScatterMode.PROMISE_IN_BOUNDS)


def _edge_agg_body(t_hbm, ep_hbm, norm_hbm, out_hbm,
                   ep_v, norm_v, rows_v, acc, g0, g1, s0, s1):
    cid = lax.axis_index("c")
    sid = lax.axis_index("s")
    nsup = jnp.where(cid == 0, _SUPA, _SUPB)
    sup_base = cid * (_NS * _SUPA) + sid * nsup

    def _zrow(i, carry):
        for g in range(_G):
            rows_v[i, pl.ds(g * 16, 16)] = jnp.zeros((16,), jnp.float32)
        return carry

    lax.fori_loop(0, 2 * _C, _zrow, 0)
    row0 = sid * _RPT
    for k in range(_RPT // (2 * _C)):
        pltpu.sync_copy(rows_v, acc.at[pl.ds(row0 + k * 2 * _C, 2 * _C)])
    plsc.subcore_barrier()

    h0 = rows_v.at[pl.ds(0, _C)]
    h1 = rows_v.at[pl.ds(_C, _C)]

    def _scale(half_ref, c, carry_unused):
        def _sub(j, c2):
            jr = j * 16
            ng = norm_v[pl.ds(c * _C + jr, 16)]
            for lane in range(16):
                nv = _bcast_lane(ng, lane)
                for g in range(_G):
                    s = pl.ds(g * 16, 16)
                    half_ref[jr + lane, s] = half_ref[jr + lane, s] * nv
            return c2

        return lax.fori_loop(0, _C // 16, _sub, carry_unused)

    def _super(sup, carry_sup):
        pltpu.sync_copy(ep_hbm.at[sup_base + sup], ep_v)
        pltpu.sync_copy(
            norm_hbm.at[pl.ds((sup_base + sup) * _SUPE, _SUPE)], norm_v)
        pltpu.async_copy(t_hbm.at[ep_v.at[0, 0]], h0, g0)

        def _pair(k, carry):
            ca = 2 * k
            cb = 2 * k + 1
            pltpu.make_async_copy(t_hbm.at[ep_v.at[0, ca]], h0, g0).wait()
            _scale(h0, ca, 0)

            @pl.when(k > 0)
            def _():
                pltpu.make_async_copy(h1, acc.at[ep_v.at[1, ca - 1]], s1).wait()

            pltpu.async_copy(t_hbm.at[ep_v.at[0, cb]], h1, g1)
            pltpu.async_copy(h0, acc.at[ep_v.at[1, ca]], s0, add=True)
            pltpu.make_async_copy(t_hbm.at[ep_v.at[0, cb]], h1, g1).wait()
            _scale(h1, cb, 0)
            pltpu.make_async_copy(h0, acc.at[ep_v.at[1, ca]], s0).wait()

            @pl.when(k < _NCHUNK // 2 - 1)
            def _():
                pltpu.async_copy(t_hbm.at[ep_v.at[0, ca + 2]], h0, g0)

            pltpu.async_copy(h1, acc.at[ep_v.at[1, cb]], s1, add=True)
            return carry

        lax.fori_loop(0, _NCHUNK // 2, _pair, 0)
        pltpu.make_async_copy(
            h1, acc.at[ep_v.at[1, _NCHUNK - 1]], s1).wait()
        return carry_sup

    lax.fori_loop(0, nsup, _super, 0)

    plsc.subcore_barrier()
    pltpu.sync_copy(acc.at[pl.ds(row0, _RPT)],
                    out_hbm.at[cid, pl.ds(row0, _RPT)])


@functools.cache
def _edge_agg_kernel():
    mesh = plsc.VectorSubcoreMesh(core_axis_name="c", subcore_axis_name="s",
                                  num_cores=_NC, num_subcores=_NS)
    return pl.kernel(
        _edge_agg_body,
        out_type=jax.ShapeDtypeStruct((_NC, _NP, _H), jnp.float32),
        mesh=mesh,
        scratch_types=[
            pltpu.VMEM((2, _NCHUNK, _C), jnp.int32),
            pltpu.VMEM((_SUPE,), jnp.float32),
            pltpu.VMEM((2 * _C, _H), jnp.float32),
            pltpu.VMEM_SHARED((_NP, _H), jnp.float32),
            pltpu.SemaphoreType.DMA,
            pltpu.SemaphoreType.DMA,
            pltpu.SemaphoreType.DMA,
            pltpu.SemaphoreType.DMA,
        ],
    )


def kernel(h, edge_index, r, norm, embed, weights, loop_weights, biases):
    x = jnp.take(embed, h, axis=0)

    ii = jnp.arange(_NB)
    wd = jnp.zeros((_NL, _R, _NB, _SUB, _NB, _SUB), jnp.float32)
    wd = wd.at[:, :, ii, :, ii, :].set(jnp.transpose(weights, (2, 0, 1, 3, 4)))
    wd = wd.reshape(_NL, _R, _H, _H)

    pad = _EPAD - _E

    def _pad1(a, dtype):
        a = a.astype(dtype).reshape(-1)
        return jnp.concatenate([a, jnp.zeros((pad,), dtype)])

    gidx = _pad1(r.astype(jnp.int32) * _N + edge_index[0], jnp.int32)
    dst2 = _pad1(edge_index[1], jnp.int32)
    ep = jnp.stack([gidx.reshape(_TSUP, _NCHUNK, _C),
                    dst2.reshape(_TSUP, _NCHUNK, _C)], axis=1)
    norm2 = _pad1(norm, jnp.float32)

    t, sl = _transform(x, wd[0], loop_weights[0], biases[0].reshape(1, _H))
    p = _edge_agg_kernel()(t.reshape(_R * _N, _H), ep, norm2)
    for l in range(1, _NL):
        t, sl = _transform_fused(p, sl, wd[l], loop_weights[l],
                                 biases[l].reshape(1, _H))
        p = _edge_agg_kernel()(t.reshape(_R * _N, _H), ep, norm2)
    return _combine(p, sl, relu=False)
